# baseline (speedup 1.0000x reference)
import functools

import numpy as np

import jax
import jax.numpy as jnp
from jax import lax
from jax.experimental import pallas as pl
from jax.experimental.pallas import tpu as pltpu
from jax.experimental.pallas import tpu_sc as plsc

B, S = 1024, 200
T = B * S
N_NUM = 16
D = 128
NC, NS = 2, 16
NW = NC * NS
TPW = T // NW
CHUNK = 128
NCHUNK = TPW // CHUNK
C_ROWS = 512 + 4096


NSET = 3


def _sc_gather(pt_i, pr_i, bb_i, st_i, pth_i, pa_i, ctab):
    mesh = plsc.VectorSubcoreMesh(core_axis_name="c", subcore_axis_name="s")

    @functools.partial(
        pl.kernel,
        mesh=mesh,
        out_type=jax.ShapeDtypeStruct((T, D), jnp.float32),
        scratch_types=[
            pltpu.VMEM((TPW,), jnp.int32),
            pltpu.VMEM((TPW,), jnp.int32),
            pltpu.VMEM((TPW,), jnp.int32),
            pltpu.VMEM((NCHUNK, CHUNK), jnp.int32),
            pltpu.VMEM((NCHUNK, CHUNK), jnp.int32),
            [pltpu.VMEM((CHUNK, D), jnp.float32) for _ in range(NSET)],
            [pltpu.SemaphoreType.DMA for _ in range(NSET)],
            [pltpu.SemaphoreType.DMA for _ in range(NSET)],
            [pltpu.SemaphoreType.DMA for _ in range(NSET)],
            pltpu.VMEM_SHARED((C_ROWS, D), jnp.float32),
        ],
    )
    def k(pt_h, pr_h, bb_h, st_h, pth_h, pa_h, c_h, out_h,
          ida, idb, idc, i1_v, i2_v, gset, s_init, s_add, s_out,
          c_sh):
        wid = lax.axis_index("s") * NC + lax.axis_index("c")
        base = wid * TPW

        @pl.when(lax.axis_index("s") == 0)
        def _():
            pltpu.sync_copy(c_h, c_sh)

        plsc.subcore_barrier()

        pltpu.sync_copy(pt_h.at[pl.ds(base, TPW)], ida)
        pltpu.sync_copy(pr_h.at[pl.ds(base, TPW)], idb)

        def bd1(c, carry):
            for j in range(CHUNK // 16):
                o = c * CHUNK + j * 16
                v = ida[pl.ds(o, 16)] * 16 + idb[pl.ds(o, 16)]
                i1_v[c, pl.ds(j * 16, 16)] = v
            return carry

        lax.fori_loop(0, NCHUNK, bd1, 0)
        pltpu.sync_copy(bb_h.at[pl.ds(base, TPW)], ida)
        pltpu.sync_copy(st_h.at[pl.ds(base, TPW)], idb)
        pltpu.sync_copy(pth_h.at[pl.ds(base, TPW)], idc)

        def bd2(c, carry):
            for j in range(CHUNK // 16):
                o = c * CHUNK + j * 16
                v = (ida[pl.ds(o, 16)] * 16 + idb[pl.ds(o, 16)] * 4
                     + idc[pl.ds(o, 16)])
                i2_v[c, pl.ds(j * 16, 16)] = v
            return carry

        lax.fori_loop(0, NCHUNK, bd2, 0)
        pltpu.sync_copy(pa_h.at[pl.ds(base, TPW)], ida)

        def bd3(c, carry):
            for j in range(CHUNK // 16):
                o = c * CHUNK + j * 16
                sl = pl.ds(j * 16, 16)
                i2_v[c, sl] = i2_v[c, sl] * 32 + ida[pl.ds(o, 16)] + 512
            return carry

        lax.fori_loop(0, NCHUNK, bd3, 0)

        cps = {}
        for i in range(NCHUNK + 2):
            if i < NCHUNK:
                s = i % NSET
                if i >= NSET:
                    cps[("out", i - NSET)].wait()
                cps[("init", i)] = pltpu.async_copy(
                    c_sh.at[i1_v.at[i]], gset[s], s_init[s])
            j = i - 1
            if 0 <= j < NCHUNK:
                s = j % NSET
                cps[("init", j)].wait()
                cps[("a2", j)] = pltpu.async_copy(
                    c_sh.at[i2_v.at[j]], gset[s], s_add[s], add=True)
            q = i - 2
            if 0 <= q < NCHUNK:
                s = q % NSET
                cps[("a2", q)].wait()
                cps[("out", q)] = pltpu.async_copy(
                    gset[s], out_h.at[pl.ds(base + q * CHUNK, CHUNK)],
                    s_out[s])
        for q in range(max(0, NCHUNK - NSET), NCHUNK):
            cps[("out", q)].wait()

    return k(pt_i, pr_i, bb_i, st_i, pth_i, pa_i, ctab)


def _tc_dense(num8, msk8, g, wbig, seg, lnnw_t, lnnb_t, ball, lnw, lnb):
    BT = 2048
    BTR = BT // 8
    grid = (T // BT,)

    def body(num_ref, msk_ref, g_ref, wbig_ref, seg_ref, lnnwt_ref,
             lnnbt_ref, ball_ref, lnw_ref, lnb_ref, out_ref):
        x = num_ref[...]
        s1 = jnp.dot(x, seg_ref[...], preferred_element_type=jnp.float32)
        s2 = jnp.dot(x * x, seg_ref[...], preferred_element_type=jnp.float32)
        mean = s1 * (1.0 / N_NUM)
        var = s2 * (1.0 / N_NUM) - mean * mean
        y = (x - mean) * lax.rsqrt(var + 1e-5) * lnnwt_ref[...] + lnnbt_ref[...]
        y = y * msk_ref[...]
        z8 = jnp.dot(y, wbig_ref[...], preferred_element_type=jnp.float32)
        z = z8.reshape(BT, D) + g_ref[...].astype(jnp.float32) + ball_ref[...]
        mu2 = jnp.mean(z, axis=-1, keepdims=True)
        zc = z - mu2
        var2 = jnp.mean(zc * zc, axis=-1, keepdims=True)
        out_ref[...] = zc * lax.rsqrt(var2 + 1e-5) * lnw_ref[...] + lnb_ref[...]

    return pl.pallas_call(
        body,
        grid=grid,
        in_specs=[
            pl.BlockSpec((BTR, D), lambda i: (i, 0)),
            pl.BlockSpec((BTR, D), lambda i: (i, 0)),
            pl.BlockSpec((BT, D), lambda i: (i, 0)),
            pl.BlockSpec((D, 8 * D), lambda i: (0, 0)),
            pl.BlockSpec((D, D), lambda i: (0, 0)),
            pl.BlockSpec((1, D), lambda i: (0, 0)),
            pl.BlockSpec((1, D), lambda i: (0, 0)),
            pl.BlockSpec((1, D), lambda i: (0, 0)),
            pl.BlockSpec((1, D), lambda i: (0, 0)),
            pl.BlockSpec((1, D), lambda i: (0, 0)),
        ],
        out_specs=pl.BlockSpec((BT, D), lambda i: (i, 0)),
        out_shape=jax.ShapeDtypeStruct((T, D), jnp.float32),
    )(num8, msk8, g, wbig, seg, lnnw_t, lnnb_t, ball, lnw, lnb)


def kernel(pitch_type_ids, pitch_result_ids, bb_type_ids, stand_ids,
           p_throws_ids, pa_event_ids, numeric_features, numeric_mask,
           T_pt, T_pr, T_bb, T_st, T_pth, T_pa, ln_num_w, ln_num_b,
           W_num, b_num, W_proj, b_proj, ln_w, ln_b):
    f32 = jnp.float32
    pt_i = pitch_type_ids.reshape(T).astype(jnp.int32)
    pr_i = pitch_result_ids.reshape(T).astype(jnp.int32)
    bb_i = bb_type_ids.reshape(T).astype(jnp.int32)
    st_i = stand_ids.reshape(T).astype(jnp.int32)
    pth_i = p_throws_ids.reshape(T).astype(jnp.int32)
    pa_i = pa_event_ids.reshape(T).astype(jnp.int32)
    mskf = numeric_mask.astype(f32)
    num8 = (numeric_features * mskf).reshape(T // 8, 8 * N_NUM)
    msk8 = mskf.reshape(T // 8, 8 * N_NUM)

    P_pt = T_pt @ W_proj[:, 0:32].T
    P_pr = T_pr @ W_proj[:, 32:64].T
    P_bb = T_bb @ W_proj[:, 64:80].T
    P_st = T_st @ W_proj[:, 80:88].T
    P_pth = T_pth @ W_proj[:, 88:96].T
    P_pa = T_pa @ W_proj[:, 96:128].T
    C1 = (P_pt[:, None, :] + P_pr[None, :, :]).reshape(512, D)
    C2 = (P_bb[:, None, None, None, :] + P_st[None, :, None, None, :]
          + P_pth[None, None, :, None, :]
          + P_pa[None, None, None, :, :]).reshape(4096, D)
    Wn = W_proj[:, 128:144]
    w_eff_t = W_num.T @ Wn.T
    ball = (b_proj + b_num @ Wn.T).reshape(1, D)

    ctab = jnp.concatenate([C1, C2], axis=0)
    g = _sc_gather(pt_i, pr_i, bb_i, st_i, pth_i, pa_i, ctab)

    wbig = jnp.kron(jnp.eye(8, dtype=f32), w_eff_t)
    seg = jnp.asarray(np.kron(np.eye(8, dtype=np.float32),
                              np.ones((N_NUM, N_NUM), np.float32)))
    lnnw_t = jnp.tile(ln_num_w, 8).reshape(1, D)
    lnnb_t = jnp.tile(ln_num_b, 8).reshape(1, D)
    out = _tc_dense(num8, msk8, g, wbig, seg, lnnw_t, lnnb_t,
                    ball, ln_w.reshape(1, D), ln_b.reshape(1, D))
    return out.reshape(B, S, D)


# speedup vs baseline: 1.1039x; 1.1039x over previous
import functools

import numpy as np

import jax
import jax.numpy as jnp
from jax import lax
from jax.experimental import pallas as pl
from jax.experimental.pallas import tpu as pltpu
from jax.experimental.pallas import tpu_sc as plsc

B, S = 1024, 200
T = B * S
N_NUM = 16
D = 128
NC, NS = 2, 16
NW = NC * NS
TPW = T // NW
CHUNK = 128
NCHUNK = TPW // CHUNK
C_ROWS = 512 + 4096


NSET = 3


def _sc_gather(pt_i, pr_i, bb_i, st_i, pth_i, pa_i, ctab):
    mesh = plsc.VectorSubcoreMesh(core_axis_name="c", subcore_axis_name="s")

    @functools.partial(
        pl.kernel,
        mesh=mesh,
        out_type=jax.ShapeDtypeStruct((T, D), jnp.float32),
        scratch_types=[
            pltpu.VMEM((TPW,), jnp.int32),
            pltpu.VMEM((TPW,), jnp.int32),
            pltpu.VMEM((TPW,), jnp.int32),
            pltpu.VMEM((NCHUNK, CHUNK), jnp.int32),
            pltpu.VMEM((NCHUNK, CHUNK), jnp.int32),
            [pltpu.VMEM((CHUNK, D), jnp.float32) for _ in range(NSET)],
            [pltpu.SemaphoreType.DMA for _ in range(NSET)],
            [pltpu.SemaphoreType.DMA for _ in range(NSET)],
            [pltpu.SemaphoreType.DMA for _ in range(NSET)],
            pltpu.VMEM_SHARED((C_ROWS, D), jnp.float32),
        ],
    )
    def k(pt_h, pr_h, bb_h, st_h, pth_h, pa_h, c_h, out_h,
          ida, idb, idc, i1_v, i2_v, gset, s_init, s_add, s_out,
          c_sh):
        wid = lax.axis_index("s") * NC + lax.axis_index("c")
        base = wid * TPW

        @pl.when(lax.axis_index("s") == 0)
        def _():
            pltpu.sync_copy(c_h, c_sh)

        plsc.subcore_barrier()

        pltpu.sync_copy(pt_h.at[pl.ds(base, TPW)], ida)
        pltpu.sync_copy(pr_h.at[pl.ds(base, TPW)], idb)

        def bd1(c, carry):
            for j in range(CHUNK // 16):
                o = c * CHUNK + j * 16
                v = ida[pl.ds(o, 16)] * 16 + idb[pl.ds(o, 16)]
                i1_v[c, pl.ds(j * 16, 16)] = v
            return carry

        lax.fori_loop(0, NCHUNK, bd1, 0)
        pltpu.sync_copy(bb_h.at[pl.ds(base, TPW)], ida)
        pltpu.sync_copy(st_h.at[pl.ds(base, TPW)], idb)
        pltpu.sync_copy(pth_h.at[pl.ds(base, TPW)], idc)

        def bd2(c, carry):
            for j in range(CHUNK // 16):
                o = c * CHUNK + j * 16
                v = (ida[pl.ds(o, 16)] * 16 + idb[pl.ds(o, 16)] * 4
                     + idc[pl.ds(o, 16)])
                i2_v[c, pl.ds(j * 16, 16)] = v
            return carry

        lax.fori_loop(0, NCHUNK, bd2, 0)
        pltpu.sync_copy(pa_h.at[pl.ds(base, TPW)], ida)

        def bd3(c, carry):
            for j in range(CHUNK // 16):
                o = c * CHUNK + j * 16
                sl = pl.ds(j * 16, 16)
                i2_v[c, sl] = i2_v[c, sl] * 32 + ida[pl.ds(o, 16)] + 512
            return carry

        lax.fori_loop(0, NCHUNK, bd3, 0)

        cps = {}
        for i in range(NCHUNK + 2):
            if i < NCHUNK:
                s = i % NSET
                if i >= NSET:
                    cps[("out", i - NSET)].wait()
                cps[("init", i)] = pltpu.async_copy(
                    c_sh.at[i1_v.at[i]], gset[s], s_init[s])
            j = i - 1
            if 0 <= j < NCHUNK:
                s = j % NSET
                cps[("init", j)].wait()
                cps[("a2", j)] = pltpu.async_copy(
                    c_sh.at[i2_v.at[j]], gset[s], s_add[s], add=True)
            q = i - 2
            if 0 <= q < NCHUNK:
                s = q % NSET
                cps[("a2", q)].wait()
                cps[("out", q)] = pltpu.async_copy(
                    gset[s], out_h.at[pl.ds(base + q * CHUNK, CHUNK)],
                    s_out[s])
        for q in range(max(0, NCHUNK - NSET), NCHUNK):
            cps[("out", q)].wait()

    return k(pt_i, pr_i, bb_i, st_i, pth_i, pa_i, ctab)


def _tc_dense(num8, msk8, g, wbig, seg, lnnw_t, lnnb_t, ball, lnw, lnb):
    BT = 4096
    BTR = BT // 8
    grid = (T // BT,)

    def body(num_ref, msk_ref, g_ref, wbig_ref, seg_ref, lnnwt_ref,
             lnnbt_ref, ball_ref, lnw_ref, lnb_ref, out_ref):
        x = num_ref[...]
        s1 = jnp.dot(x, seg_ref[...], preferred_element_type=jnp.float32)
        s2 = jnp.dot(x * x, seg_ref[...], preferred_element_type=jnp.float32)
        mean = s1 * (1.0 / N_NUM)
        var = s2 * (1.0 / N_NUM) - mean * mean
        y = (x - mean) * lax.rsqrt(var + 1e-5) * lnnwt_ref[...] + lnnbt_ref[...]
        y = y * msk_ref[...]
        z8 = jnp.dot(y, wbig_ref[...], preferred_element_type=jnp.float32)
        z = z8.reshape(BT, D) + g_ref[...].astype(jnp.float32) + ball_ref[...]
        mu2 = jnp.mean(z, axis=-1, keepdims=True)
        zc = z - mu2
        var2 = jnp.mean(zc * zc, axis=-1, keepdims=True)
        out_ref[...] = zc * lax.rsqrt(var2 + 1e-5) * lnw_ref[...] + lnb_ref[...]

    return pl.pallas_call(
        body,
        grid=grid,
        in_specs=[
            pl.BlockSpec((BTR, D), lambda i: (i, 0)),
            pl.BlockSpec((BTR, D), lambda i: (i, 0)),
            pl.BlockSpec((BT, D), lambda i: (i, 0)),
            pl.BlockSpec((D, 8 * D), lambda i: (0, 0)),
            pl.BlockSpec((D, D), lambda i: (0, 0)),
            pl.BlockSpec((1, D), lambda i: (0, 0)),
            pl.BlockSpec((1, D), lambda i: (0, 0)),
            pl.BlockSpec((1, D), lambda i: (0, 0)),
            pl.BlockSpec((1, D), lambda i: (0, 0)),
            pl.BlockSpec((1, D), lambda i: (0, 0)),
        ],
        out_specs=pl.BlockSpec((BT, D), lambda i: (i, 0)),
        out_shape=jax.ShapeDtypeStruct((T, D), jnp.float32),
    )(num8, msk8, g, wbig, seg, lnnw_t, lnnb_t, ball, lnw, lnb)


def kernel(pitch_type_ids, pitch_result_ids, bb_type_ids, stand_ids,
           p_throws_ids, pa_event_ids, numeric_features, numeric_mask,
           T_pt, T_pr, T_bb, T_st, T_pth, T_pa, ln_num_w, ln_num_b,
           W_num, b_num, W_proj, b_proj, ln_w, ln_b):
    f32 = jnp.float32
    pt_i = pitch_type_ids.reshape(T).astype(jnp.int32)
    pr_i = pitch_result_ids.reshape(T).astype(jnp.int32)
    bb_i = bb_type_ids.reshape(T).astype(jnp.int32)
    st_i = stand_ids.reshape(T).astype(jnp.int32)
    pth_i = p_throws_ids.reshape(T).astype(jnp.int32)
    pa_i = pa_event_ids.reshape(T).astype(jnp.int32)
    msk8 = numeric_mask.reshape(T // 8, 8 * N_NUM).astype(f32)
    num8 = numeric_features.reshape(T // 8, 8 * N_NUM) * msk8

    P_pt = T_pt @ W_proj[:, 0:32].T
    P_pr = T_pr @ W_proj[:, 32:64].T
    P_bb = T_bb @ W_proj[:, 64:80].T
    P_st = T_st @ W_proj[:, 80:88].T
    P_pth = T_pth @ W_proj[:, 88:96].T
    P_pa = T_pa @ W_proj[:, 96:128].T
    C1 = (P_pt[:, None, :] + P_pr[None, :, :]).reshape(512, D)
    C2 = (P_bb[:, None, None, None, :] + P_st[None, :, None, None, :]
          + P_pth[None, None, :, None, :]
          + P_pa[None, None, None, :, :]).reshape(4096, D)
    Wn = W_proj[:, 128:144]
    w_eff_t = W_num.T @ Wn.T
    ball = (b_proj + b_num @ Wn.T).reshape(1, D)

    ctab = jnp.concatenate([C1, C2], axis=0)
    g = _sc_gather(pt_i, pr_i, bb_i, st_i, pth_i, pa_i, ctab)

    wbig = jnp.kron(jnp.eye(8, dtype=f32), w_eff_t)
    seg = jnp.asarray(np.kron(np.eye(8, dtype=np.float32),
                              np.ones((N_NUM, N_NUM), np.float32)))
    lnnw_t = jnp.tile(ln_num_w, 8).reshape(1, D)
    lnnb_t = jnp.tile(ln_num_b, 8).reshape(1, D)
    out = _tc_dense(num8, msk8, g, wbig, seg, lnnw_t, lnnb_t,
                    ball, ln_w.reshape(1, D), ln_b.reshape(1, D))
    return out.reshape(B, S, D)


# speedup vs baseline: 1.1612x; 1.0519x over previous
import functools

import numpy as np

import jax
import jax.numpy as jnp
from jax import lax
from jax.experimental import pallas as pl
from jax.experimental.pallas import tpu as pltpu
from jax.experimental.pallas import tpu_sc as plsc

B, S = 1024, 200
T = B * S
N_NUM = 16
D = 128
NC, NS = 2, 16
NW = NC * NS
TPW = T // NW
CHUNK = 128
NCHUNK = TPW // CHUNK
C_ROWS = 512 + 4096


NSET = 3


def _sc_gather(pt_i, pr_i, bb_i, st_i, pth_i, pa_i, ctab):
    mesh = plsc.VectorSubcoreMesh(core_axis_name="c", subcore_axis_name="s")

    @functools.partial(
        pl.kernel,
        mesh=mesh,
        out_type=jax.ShapeDtypeStruct((T, D), jnp.float32),
        scratch_types=[
            pltpu.VMEM((TPW,), jnp.int32),
            pltpu.VMEM((TPW,), jnp.int32),
            pltpu.VMEM((TPW,), jnp.int32),
            pltpu.VMEM((NCHUNK, CHUNK), jnp.int32),
            pltpu.VMEM((NCHUNK, CHUNK), jnp.int32),
            [pltpu.VMEM((CHUNK, D), jnp.float32) for _ in range(NSET)],
            [pltpu.SemaphoreType.DMA for _ in range(NSET)],
            [pltpu.SemaphoreType.DMA for _ in range(NSET)],
            [pltpu.SemaphoreType.DMA for _ in range(NSET)],
            pltpu.VMEM_SHARED((C_ROWS, D), jnp.float32),
        ],
    )
    def k(pt_h, pr_h, bb_h, st_h, pth_h, pa_h, c_h, out_h,
          ida, idb, idc, i1_v, i2_v, gset, s_init, s_add, s_out,
          c_sh):
        wid = lax.axis_index("s") * NC + lax.axis_index("c")
        base = wid * TPW

        @pl.when(lax.axis_index("s") == 0)
        def _():
            pltpu.sync_copy(c_h, c_sh)

        plsc.subcore_barrier()

        pltpu.sync_copy(pt_h.at[pl.ds(base, TPW)], ida)
        pltpu.sync_copy(pr_h.at[pl.ds(base, TPW)], idb)

        def bd1(c, carry):
            for j in range(CHUNK // 16):
                o = c * CHUNK + j * 16
                v = ida[pl.ds(o, 16)] * 16 + idb[pl.ds(o, 16)]
                i1_v[c, pl.ds(j * 16, 16)] = v
            return carry

        lax.fori_loop(0, NCHUNK, bd1, 0)
        pltpu.sync_copy(bb_h.at[pl.ds(base, TPW)], ida)
        pltpu.sync_copy(st_h.at[pl.ds(base, TPW)], idb)
        pltpu.sync_copy(pth_h.at[pl.ds(base, TPW)], idc)

        def bd2(c, carry):
            for j in range(CHUNK // 16):
                o = c * CHUNK + j * 16
                v = (ida[pl.ds(o, 16)] * 16 + idb[pl.ds(o, 16)] * 4
                     + idc[pl.ds(o, 16)])
                i2_v[c, pl.ds(j * 16, 16)] = v
            return carry

        lax.fori_loop(0, NCHUNK, bd2, 0)
        pltpu.sync_copy(pa_h.at[pl.ds(base, TPW)], ida)

        def bd3(c, carry):
            for j in range(CHUNK // 16):
                o = c * CHUNK + j * 16
                sl = pl.ds(j * 16, 16)
                i2_v[c, sl] = i2_v[c, sl] * 32 + ida[pl.ds(o, 16)] + 512
            return carry

        lax.fori_loop(0, NCHUNK, bd3, 0)

        cps = {}
        for i in range(NCHUNK + 2):
            if i < NCHUNK:
                s = i % NSET
                if i >= NSET:
                    cps[("out", i - NSET)].wait()
                cps[("init", i)] = pltpu.async_copy(
                    c_sh.at[i1_v.at[i]], gset[s], s_init[s])
            j = i - 1
            if 0 <= j < NCHUNK:
                s = j % NSET
                cps[("init", j)].wait()
                cps[("a2", j)] = pltpu.async_copy(
                    c_sh.at[i2_v.at[j]], gset[s], s_add[s], add=True)
            q = i - 2
            if 0 <= q < NCHUNK:
                s = q % NSET
                cps[("a2", q)].wait()
                cps[("out", q)] = pltpu.async_copy(
                    gset[s], out_h.at[pl.ds(base + q * CHUNK, CHUNK)],
                    s_out[s])
        for q in range(max(0, NCHUNK - NSET), NCHUNK):
            cps[("out", q)].wait()

    return k(pt_i, pr_i, bb_i, st_i, pth_i, pa_i, ctab)


def _tc_dense(num8, msk8, g, wbig, seg, lnnw_t, lnnb_t, ball, lnw, lnb):
    BT = 4096
    BTR = BT // 8
    grid = (T // BT,)

    def body(num_ref, msk_ref, g_ref, wbig_ref, seg_ref, lnnwt_ref,
             lnnbt_ref, ball_ref, lnw_ref, lnb_ref, out_ref):
        m = msk_ref[...].astype(jnp.float32)
        x = num_ref[...] * m
        s1 = jnp.dot(x, seg_ref[...], preferred_element_type=jnp.float32)
        s2 = jnp.dot(x * x, seg_ref[...], preferred_element_type=jnp.float32)
        mean = s1 * (1.0 / N_NUM)
        var = s2 * (1.0 / N_NUM) - mean * mean
        y = (x - mean) * lax.rsqrt(var + 1e-5) * lnnwt_ref[...] + lnnbt_ref[...]
        y = y * m
        z8 = jnp.dot(y, wbig_ref[...], preferred_element_type=jnp.float32)
        z = z8.reshape(BT, D) + g_ref[...].astype(jnp.float32) + ball_ref[...]
        mu2 = jnp.mean(z, axis=-1, keepdims=True)
        zc = z - mu2
        var2 = jnp.mean(zc * zc, axis=-1, keepdims=True)
        out_ref[...] = zc * lax.rsqrt(var2 + 1e-5) * lnw_ref[...] + lnb_ref[...]

    return pl.pallas_call(
        body,
        grid=grid,
        in_specs=[
            pl.BlockSpec((BTR, D), lambda i: (i, 0)),
            pl.BlockSpec((BTR, D), lambda i: (i, 0)),
            pl.BlockSpec((BT, D), lambda i: (i, 0)),
            pl.BlockSpec((D, 8 * D), lambda i: (0, 0)),
            pl.BlockSpec((D, D), lambda i: (0, 0)),
            pl.BlockSpec((1, D), lambda i: (0, 0)),
            pl.BlockSpec((1, D), lambda i: (0, 0)),
            pl.BlockSpec((1, D), lambda i: (0, 0)),
            pl.BlockSpec((1, D), lambda i: (0, 0)),
            pl.BlockSpec((1, D), lambda i: (0, 0)),
        ],
        out_specs=pl.BlockSpec((BT, D), lambda i: (i, 0)),
        out_shape=jax.ShapeDtypeStruct((T, D), jnp.float32),
    )(num8, msk8, g, wbig, seg, lnnw_t, lnnb_t, ball, lnw, lnb)


def kernel(pitch_type_ids, pitch_result_ids, bb_type_ids, stand_ids,
           p_throws_ids, pa_event_ids, numeric_features, numeric_mask,
           T_pt, T_pr, T_bb, T_st, T_pth, T_pa, ln_num_w, ln_num_b,
           W_num, b_num, W_proj, b_proj, ln_w, ln_b):
    f32 = jnp.float32
    pt_i = pitch_type_ids.reshape(T).astype(jnp.int32)
    pr_i = pitch_result_ids.reshape(T).astype(jnp.int32)
    bb_i = bb_type_ids.reshape(T).astype(jnp.int32)
    st_i = stand_ids.reshape(T).astype(jnp.int32)
    pth_i = p_throws_ids.reshape(T).astype(jnp.int32)
    pa_i = pa_event_ids.reshape(T).astype(jnp.int32)
    msk8 = numeric_mask.astype(jnp.int8).reshape(T // 8, 8 * N_NUM)
    num8 = numeric_features.reshape(T // 8, 8 * N_NUM)

    P_pt = T_pt @ W_proj[:, 0:32].T
    P_pr = T_pr @ W_proj[:, 32:64].T
    P_bb = T_bb @ W_proj[:, 64:80].T
    P_st = T_st @ W_proj[:, 80:88].T
    P_pth = T_pth @ W_proj[:, 88:96].T
    P_pa = T_pa @ W_proj[:, 96:128].T
    C1 = (P_pt[:, None, :] + P_pr[None, :, :]).reshape(512, D)
    C2 = (P_bb[:, None, None, None, :] + P_st[None, :, None, None, :]
          + P_pth[None, None, :, None, :]
          + P_pa[None, None, None, :, :]).reshape(4096, D)
    Wn = W_proj[:, 128:144]
    w_eff_t = W_num.T @ Wn.T
    ball = (b_proj + b_num @ Wn.T).reshape(1, D)

    ctab = jnp.concatenate([C1, C2], axis=0)
    g = _sc_gather(pt_i, pr_i, bb_i, st_i, pth_i, pa_i, ctab)

    wbig = jnp.kron(jnp.eye(8, dtype=f32), w_eff_t)
    seg = jnp.asarray(np.kron(np.eye(8, dtype=np.float32),
                              np.ones((N_NUM, N_NUM), np.float32)))
    lnnw_t = jnp.tile(ln_num_w, 8).reshape(1, D)
    lnnb_t = jnp.tile(ln_num_b, 8).reshape(1, D)
    out = _tc_dense(num8, msk8, g, wbig, seg, lnnw_t, lnnb_t,
                    ball, ln_w.reshape(1, D), ln_b.reshape(1, D))
    return out.reshape(B, S, D)


# speedup vs baseline: 1.1710x; 1.0085x over previous
import functools

import numpy as np

import jax
import jax.numpy as jnp
from jax import lax
from jax.experimental import pallas as pl
from jax.experimental.pallas import tpu as pltpu
from jax.experimental.pallas import tpu_sc as plsc

B, S = 1024, 200
T = B * S
N_NUM = 16
D = 128
NC, NS = 2, 16
NW = NC * NS
TPW = T // NW
CHUNK = 128
NCHUNK = TPW // CHUNK
C_ROWS = 512 + 4096


NSET = 3


def _sc_gather(pt_i, pr_i, bb_i, st_i, pth_i, pa_i, ctab):
    mesh = plsc.VectorSubcoreMesh(core_axis_name="c", subcore_axis_name="s")

    @functools.partial(
        pl.kernel,
        mesh=mesh,
        out_type=jax.ShapeDtypeStruct((T, D), jnp.float32),
        scratch_types=[
            pltpu.VMEM((TPW,), jnp.int32),
            pltpu.VMEM((TPW,), jnp.int32),
            pltpu.VMEM((TPW,), jnp.int32),
            pltpu.VMEM((NCHUNK, CHUNK), jnp.int32),
            pltpu.VMEM((NCHUNK, CHUNK), jnp.int32),
            [pltpu.VMEM((CHUNK, D), jnp.float32) for _ in range(NSET)],
            [pltpu.SemaphoreType.DMA for _ in range(NSET)],
            [pltpu.SemaphoreType.DMA for _ in range(NSET)],
            [pltpu.SemaphoreType.DMA for _ in range(NSET)],
            pltpu.VMEM_SHARED((C_ROWS, D), jnp.float32),
        ],
    )
    def k(pt_h, pr_h, bb_h, st_h, pth_h, pa_h, c_h, out_h,
          ida, idb, idc, i1_v, i2_v, gset, s_init, s_add, s_out,
          c_sh):
        wid = lax.axis_index("s") * NC + lax.axis_index("c")
        base = wid * TPW

        @pl.when(lax.axis_index("s") == 0)
        def _():
            pltpu.sync_copy(c_h, c_sh)

        plsc.subcore_barrier()

        pltpu.sync_copy(pt_h.at[pl.ds(base, TPW)], ida)
        pltpu.sync_copy(pr_h.at[pl.ds(base, TPW)], idb)

        def bd1(c, carry):
            for j in range(CHUNK // 16):
                o = c * CHUNK + j * 16
                v = ida[pl.ds(o, 16)] * 16 + idb[pl.ds(o, 16)]
                i1_v[c, pl.ds(j * 16, 16)] = v
            return carry

        lax.fori_loop(0, NCHUNK, bd1, 0)
        pltpu.sync_copy(bb_h.at[pl.ds(base, TPW)], ida)
        pltpu.sync_copy(st_h.at[pl.ds(base, TPW)], idb)
        pltpu.sync_copy(pth_h.at[pl.ds(base, TPW)], idc)

        def bd2(c, carry):
            for j in range(CHUNK // 16):
                o = c * CHUNK + j * 16
                v = (ida[pl.ds(o, 16)] * 16 + idb[pl.ds(o, 16)] * 4
                     + idc[pl.ds(o, 16)])
                i2_v[c, pl.ds(j * 16, 16)] = v
            return carry

        lax.fori_loop(0, NCHUNK, bd2, 0)
        pltpu.sync_copy(pa_h.at[pl.ds(base, TPW)], ida)

        def bd3(c, carry):
            for j in range(CHUNK // 16):
                o = c * CHUNK + j * 16
                sl = pl.ds(j * 16, 16)
                i2_v[c, sl] = i2_v[c, sl] * 32 + ida[pl.ds(o, 16)] + 512
            return carry

        lax.fori_loop(0, NCHUNK, bd3, 0)

        cps = {}
        for i in range(NCHUNK + 2):
            if i < NCHUNK:
                s = i % NSET
                if i >= NSET:
                    cps[("out", i - NSET)].wait()
                cps[("init", i)] = pltpu.async_copy(
                    c_sh.at[i1_v.at[i]], gset[s], s_init[s])
            j = i - 1
            if 0 <= j < NCHUNK:
                s = j % NSET
                cps[("init", j)].wait()
                cps[("a2", j)] = pltpu.async_copy(
                    c_sh.at[i2_v.at[j]], gset[s], s_add[s], add=True)
            q = i - 2
            if 0 <= q < NCHUNK:
                s = q % NSET
                cps[("a2", q)].wait()
                cps[("out", q)] = pltpu.async_copy(
                    gset[s], out_h.at[pl.ds(base + q * CHUNK, CHUNK)],
                    s_out[s])
        for q in range(max(0, NCHUNK - NSET), NCHUNK):
            cps[("out", q)].wait()

    return k(pt_i, pr_i, bb_i, st_i, pth_i, pa_i, ctab)


def _tc_dense(num8, msk8, g, wbig, seg, lnnw_t, lnnb_t, ball, lnw, lnb):
    BT = 4096
    BTR = BT // 8
    grid = (T // BT,)

    def body(num_ref, msk_ref, g_ref, wbig_ref, seg_ref, lnnwt_ref,
             lnnbt_ref, ball_ref, lnw_ref, lnb_ref, out_ref):
        m = msk_ref[...].astype(jnp.float32)
        x = num_ref[...].astype(jnp.float32) * m
        s1 = jnp.dot(x, seg_ref[...], preferred_element_type=jnp.float32)
        s2 = jnp.dot(x * x, seg_ref[...], preferred_element_type=jnp.float32)
        mean = s1 * (1.0 / N_NUM)
        var = s2 * (1.0 / N_NUM) - mean * mean
        y = (x - mean) * lax.rsqrt(var + 1e-5) * lnnwt_ref[...] + lnnbt_ref[...]
        y = y * m
        z8 = jnp.dot(y, wbig_ref[...], preferred_element_type=jnp.float32)
        z = z8.reshape(BT, D) + g_ref[...].astype(jnp.float32) + ball_ref[...]
        mu2 = jnp.mean(z, axis=-1, keepdims=True)
        zc = z - mu2
        var2 = jnp.mean(zc * zc, axis=-1, keepdims=True)
        out_ref[...] = zc * lax.rsqrt(var2 + 1e-5) * lnw_ref[...] + lnb_ref[...]

    return pl.pallas_call(
        body,
        grid=grid,
        in_specs=[
            pl.BlockSpec((BTR, D), lambda i: (i, 0)),
            pl.BlockSpec((BTR, D), lambda i: (i, 0)),
            pl.BlockSpec((BT, D), lambda i: (i, 0)),
            pl.BlockSpec((D, 8 * D), lambda i: (0, 0)),
            pl.BlockSpec((D, D), lambda i: (0, 0)),
            pl.BlockSpec((1, D), lambda i: (0, 0)),
            pl.BlockSpec((1, D), lambda i: (0, 0)),
            pl.BlockSpec((1, D), lambda i: (0, 0)),
            pl.BlockSpec((1, D), lambda i: (0, 0)),
            pl.BlockSpec((1, D), lambda i: (0, 0)),
        ],
        out_specs=pl.BlockSpec((BT, D), lambda i: (i, 0)),
        out_shape=jax.ShapeDtypeStruct((T, D), jnp.float32),
    )(num8, msk8, g, wbig, seg, lnnw_t, lnnb_t, ball, lnw, lnb)


def kernel(pitch_type_ids, pitch_result_ids, bb_type_ids, stand_ids,
           p_throws_ids, pa_event_ids, numeric_features, numeric_mask,
           T_pt, T_pr, T_bb, T_st, T_pth, T_pa, ln_num_w, ln_num_b,
           W_num, b_num, W_proj, b_proj, ln_w, ln_b):
    f32 = jnp.float32
    pt_i = pitch_type_ids.reshape(T).astype(jnp.int32)
    pr_i = pitch_result_ids.reshape(T).astype(jnp.int32)
    bb_i = bb_type_ids.reshape(T).astype(jnp.int32)
    st_i = stand_ids.reshape(T).astype(jnp.int32)
    pth_i = p_throws_ids.reshape(T).astype(jnp.int32)
    pa_i = pa_event_ids.reshape(T).astype(jnp.int32)
    msk8 = numeric_mask.astype(jnp.int8).reshape(T // 8, 8 * N_NUM)
    num8 = numeric_features.astype(jnp.bfloat16).reshape(T // 8, 8 * N_NUM)

    P_pt = T_pt @ W_proj[:, 0:32].T
    P_pr = T_pr @ W_proj[:, 32:64].T
    P_bb = T_bb @ W_proj[:, 64:80].T
    P_st = T_st @ W_proj[:, 80:88].T
    P_pth = T_pth @ W_proj[:, 88:96].T
    P_pa = T_pa @ W_proj[:, 96:128].T
    C1 = (P_pt[:, None, :] + P_pr[None, :, :]).reshape(512, D)
    C2 = (P_bb[:, None, None, None, :] + P_st[None, :, None, None, :]
          + P_pth[None, None, :, None, :]
          + P_pa[None, None, None, :, :]).reshape(4096, D)
    Wn = W_proj[:, 128:144]
    w_eff_t = W_num.T @ Wn.T
    ball = (b_proj + b_num @ Wn.T).reshape(1, D)

    ctab = jnp.concatenate([C1, C2], axis=0)
    g = _sc_gather(pt_i, pr_i, bb_i, st_i, pth_i, pa_i, ctab)

    wbig = jnp.kron(jnp.eye(8, dtype=f32), w_eff_t)
    seg = jnp.asarray(np.kron(np.eye(8, dtype=np.float32),
                              np.ones((N_NUM, N_NUM), np.float32)))
    lnnw_t = jnp.tile(ln_num_w, 8).reshape(1, D)
    lnnb_t = jnp.tile(ln_num_b, 8).reshape(1, D)
    out = _tc_dense(num8, msk8, g, wbig, seg, lnnw_t, lnnb_t,
                    ball, ln_w.reshape(1, D), ln_b.reshape(1, D))
    return out.reshape(B, S, D)


# speedup vs baseline: 1.2362x; 1.0557x over previous
import functools

import numpy as np

import jax
import jax.numpy as jnp
from jax import lax
from jax.experimental import pallas as pl
from jax.experimental.pallas import tpu as pltpu
from jax.experimental.pallas import tpu_sc as plsc

B, S = 1024, 200
T = B * S
N_NUM = 16
D = 128
NC, NS = 2, 16
NW = NC * NS
TPW = T // NW
CHUNK = 128
NCHUNK = TPW // CHUNK
C_ROWS = 512 + 4096


NSET = 3


def _sc_gather(pt_i, pr_i, bb_i, st_i, pth_i, pa_i, ctab):
    mesh = plsc.VectorSubcoreMesh(core_axis_name="c", subcore_axis_name="s")

    @functools.partial(
        pl.kernel,
        mesh=mesh,
        out_type=jax.ShapeDtypeStruct((T, D), jnp.float32),
        scratch_types=[
            pltpu.VMEM((TPW,), jnp.int32),
            pltpu.VMEM((TPW,), jnp.int32),
            pltpu.VMEM((TPW,), jnp.int32),
            pltpu.VMEM((NCHUNK, CHUNK), jnp.int32),
            pltpu.VMEM((NCHUNK, CHUNK), jnp.int32),
            [pltpu.VMEM((CHUNK, D), jnp.float32) for _ in range(NSET)],
            [pltpu.SemaphoreType.DMA for _ in range(NSET)],
            [pltpu.SemaphoreType.DMA for _ in range(NSET)],
            [pltpu.SemaphoreType.DMA for _ in range(NSET)],
            pltpu.VMEM_SHARED((C_ROWS, D), jnp.float32),
        ],
    )
    def k(pt_h, pr_h, bb_h, st_h, pth_h, pa_h, c_h, out_h,
          ida, idb, idc, i1_v, i2_v, gset, s_init, s_add, s_out,
          c_sh):
        wid = lax.axis_index("s") * NC + lax.axis_index("c")
        base = wid * TPW

        @pl.when(lax.axis_index("s") == 0)
        def _():
            pltpu.sync_copy(c_h, c_sh)

        plsc.subcore_barrier()

        pltpu.sync_copy(pt_h.at[pl.ds(base, TPW)], ida)
        pltpu.sync_copy(pr_h.at[pl.ds(base, TPW)], idb)

        def bd1(c, carry):
            for j in range(CHUNK // 16):
                o = c * CHUNK + j * 16
                v = ida[pl.ds(o, 16)] * 16 + idb[pl.ds(o, 16)]
                i1_v[c, pl.ds(j * 16, 16)] = v
            return carry

        lax.fori_loop(0, NCHUNK, bd1, 0)
        pltpu.sync_copy(bb_h.at[pl.ds(base, TPW)], ida)
        pltpu.sync_copy(st_h.at[pl.ds(base, TPW)], idb)
        pltpu.sync_copy(pth_h.at[pl.ds(base, TPW)], idc)

        def bd2(c, carry):
            for j in range(CHUNK // 16):
                o = c * CHUNK + j * 16
                v = (ida[pl.ds(o, 16)] * 16 + idb[pl.ds(o, 16)] * 4
                     + idc[pl.ds(o, 16)])
                i2_v[c, pl.ds(j * 16, 16)] = v
            return carry

        lax.fori_loop(0, NCHUNK, bd2, 0)
        pltpu.sync_copy(pa_h.at[pl.ds(base, TPW)], ida)

        def bd3(c, carry):
            for j in range(CHUNK // 16):
                o = c * CHUNK + j * 16
                sl = pl.ds(j * 16, 16)
                i2_v[c, sl] = i2_v[c, sl] * 32 + ida[pl.ds(o, 16)] + 512
            return carry

        lax.fori_loop(0, NCHUNK, bd3, 0)

        cps = {}
        for i in range(NCHUNK + 2):
            if i < NCHUNK:
                s = i % NSET
                if i >= NSET:
                    cps[("out", i - NSET)].wait()
                cps[("init", i)] = pltpu.async_copy(
                    c_sh.at[i1_v.at[i]], gset[s], s_init[s])
            j = i - 1
            if 0 <= j < NCHUNK:
                s = j % NSET
                cps[("init", j)].wait()
                cps[("a2", j)] = pltpu.async_copy(
                    c_sh.at[i2_v.at[j]], gset[s], s_add[s], add=True)
            q = i - 2
            if 0 <= q < NCHUNK:
                s = q % NSET
                cps[("a2", q)].wait()
                cps[("out", q)] = pltpu.async_copy(
                    gset[s], out_h.at[pl.ds(base + q * CHUNK, CHUNK)],
                    s_out[s])
        for q in range(max(0, NCHUNK - NSET), NCHUNK):
            cps[("out", q)].wait()

    return k(pt_i, pr_i, bb_i, st_i, pth_i, pa_i, ctab)


def _tc_dense(num8, msk8, g, wbig, seg, lnnw_t, lnnb_t, ball, lnw, lnb):
    BT = 8192
    BTR = BT // 8
    grid = (T // BT,)

    def body(num_ref, msk_ref, g_ref, wbig_ref, seg_ref, lnnwt_ref,
             lnnbt_ref, ball_ref, lnw_ref, lnb_ref, out_ref):
        m = msk_ref[...].astype(jnp.float32)
        x = num_ref[...].astype(jnp.float32) * m
        s1 = jnp.dot(x, seg_ref[...], preferred_element_type=jnp.float32)
        s2 = jnp.dot(x * x, seg_ref[...], preferred_element_type=jnp.float32)
        mean = s1 * (1.0 / N_NUM)
        var = s2 * (1.0 / N_NUM) - mean * mean
        y = (x - mean) * lax.rsqrt(var + 1e-5) * lnnwt_ref[...] + lnnbt_ref[...]
        y = y * m
        z8 = jnp.dot(y, wbig_ref[...], preferred_element_type=jnp.float32)
        z = z8.reshape(BT, D) + g_ref[...].astype(jnp.float32) + ball_ref[...]
        mu2 = jnp.mean(z, axis=-1, keepdims=True)
        zc = z - mu2
        var2 = jnp.mean(zc * zc, axis=-1, keepdims=True)
        out_ref[...] = zc * lax.rsqrt(var2 + 1e-5) * lnw_ref[...] + lnb_ref[...]

    return pl.pallas_call(
        body,
        grid=grid,
        in_specs=[
            pl.BlockSpec((BTR, D), lambda i: (i, 0)),
            pl.BlockSpec((BTR, D), lambda i: (i, 0)),
            pl.BlockSpec((BT, D), lambda i: (i, 0)),
            pl.BlockSpec((D, 8 * D), lambda i: (0, 0)),
            pl.BlockSpec((D, D), lambda i: (0, 0)),
            pl.BlockSpec((1, D), lambda i: (0, 0)),
            pl.BlockSpec((1, D), lambda i: (0, 0)),
            pl.BlockSpec((1, D), lambda i: (0, 0)),
            pl.BlockSpec((1, D), lambda i: (0, 0)),
            pl.BlockSpec((1, D), lambda i: (0, 0)),
        ],
        out_specs=pl.BlockSpec((BT, D), lambda i: (i, 0)),
        out_shape=jax.ShapeDtypeStruct((T, D), jnp.float32),
    )(num8, msk8, g, wbig, seg, lnnw_t, lnnb_t, ball, lnw, lnb)


def kernel(pitch_type_ids, pitch_result_ids, bb_type_ids, stand_ids,
           p_throws_ids, pa_event_ids, numeric_features, numeric_mask,
           T_pt, T_pr, T_bb, T_st, T_pth, T_pa, ln_num_w, ln_num_b,
           W_num, b_num, W_proj, b_proj, ln_w, ln_b):
    f32 = jnp.float32
    pt_i = pitch_type_ids.reshape(T).astype(jnp.int32)
    pr_i = pitch_result_ids.reshape(T).astype(jnp.int32)
    bb_i = bb_type_ids.reshape(T).astype(jnp.int32)
    st_i = stand_ids.reshape(T).astype(jnp.int32)
    pth_i = p_throws_ids.reshape(T).astype(jnp.int32)
    pa_i = pa_event_ids.reshape(T).astype(jnp.int32)
    msk8 = numeric_mask.astype(jnp.int8).reshape(T // 8, 8 * N_NUM)
    num8 = numeric_features.astype(jnp.bfloat16).reshape(T // 8, 8 * N_NUM)

    P_pt = T_pt @ W_proj[:, 0:32].T
    P_pr = T_pr @ W_proj[:, 32:64].T
    P_bb = T_bb @ W_proj[:, 64:80].T
    P_st = T_st @ W_proj[:, 80:88].T
    P_pth = T_pth @ W_proj[:, 88:96].T
    P_pa = T_pa @ W_proj[:, 96:128].T
    C1 = (P_pt[:, None, :] + P_pr[None, :, :]).reshape(512, D)
    C2 = (P_bb[:, None, None, None, :] + P_st[None, :, None, None, :]
          + P_pth[None, None, :, None, :]
          + P_pa[None, None, None, :, :]).reshape(4096, D)
    Wn = W_proj[:, 128:144]
    w_eff_t = W_num.T @ Wn.T
    ball = (b_proj + b_num @ Wn.T).reshape(1, D)

    ctab = jnp.concatenate([C1, C2], axis=0)
    g = _sc_gather(pt_i, pr_i, bb_i, st_i, pth_i, pa_i, ctab)

    wbig = jnp.kron(jnp.eye(8, dtype=f32), w_eff_t)
    seg = jnp.asarray(np.kron(np.eye(8, dtype=np.float32),
                              np.ones((N_NUM, N_NUM), np.float32)))
    lnnw_t = jnp.tile(ln_num_w, 8).reshape(1, D)
    lnnb_t = jnp.tile(ln_num_b, 8).reshape(1, D)
    out = _tc_dense(num8, msk8, g, wbig, seg, lnnw_t, lnnb_t,
                    ball, ln_w.reshape(1, D), ln_b.reshape(1, D))
    return out.reshape(B, S, D)
